# Initial kernel scaffold; baseline (speedup 1.0000x reference)
import jax
import jax.numpy as jnp
from jax.experimental import pallas as pl


def kernel(cond0, cond1, cond4, cond5, emb_table, W_meta, b_meta, ln_w, ln_b):
    raise NotImplementedError("write your pallas kernel here")


# speedup vs baseline: 1.4241x; 1.4241x over previous
import jax
import jax.numpy as jnp
from jax.experimental import pallas as pl


_BLK = 2048


def _body(c0_ref, c1_ref, c4_ref, c5_ref, emb_ref, wt_ref, bm_ref, lnw_ref,
          lnb_ref, out_ref):
    c0 = c0_ref[...]
    c1 = c1_ref[...]
    c4 = c4_ref[...]
    c5 = c5_ref[...]
    e0 = emb_ref[0:1, :]
    e1 = emb_ref[1:2, :]
    w0 = wt_ref[0:1, :]
    w1 = wt_ref[1:2, :]
    bm = bm_ref[...]
    lnw = lnw_ref[...]
    lnb = lnb_ref[...]

    c1v = e0 + c1 * (e1 - e0)

    meta = c4 * w0 + c5 * w1 + bm

    mu = jnp.mean(meta, axis=-1, keepdims=True)
    xc = meta - mu
    var = jnp.mean(xc * xc, axis=-1, keepdims=True)
    meta_ln = xc * jax.lax.rsqrt(var + 1e-5) * lnw + lnb

    out_ref[...] = c0 + c1v + meta_ln


def kernel(cond0, cond1, cond4, cond5, emb_table, W_meta, b_meta, ln_w, ln_b):
    B, D = cond0.shape
    c1f = cond1.astype(jnp.float32).reshape(B, 1)
    wt = W_meta.T
    bm = b_meta.reshape(1, D)
    lnw = ln_w.reshape(1, D)
    lnb = ln_b.reshape(1, D)

    grid = (B // _BLK,)
    bspec_b = lambda width: pl.BlockSpec((_BLK, width), lambda i: (i, 0))
    bspec_w = lambda shape: pl.BlockSpec(shape, lambda i: (0, 0))

    return pl.pallas_call(
        _body,
        grid=grid,
        in_specs=[
            bspec_b(D),
            bspec_b(1),
            bspec_b(1),
            bspec_b(1),
            bspec_w((2, D)),
            bspec_w((2, D)),
            bspec_w((1, D)),
            bspec_w((1, D)),
            bspec_w((1, D)),
        ],
        out_specs=bspec_b(D),
        out_shape=jax.ShapeDtypeStruct((B, D), jnp.float32),
    )(cond0, c1f, cond4, cond5, emb_table, wt, bm, lnw, lnb)
